# baseline (speedup 1.0000x reference)
import functools

import jax
import jax.numpy as jnp
from jax import lax
from jax.experimental import pallas as pl
from jax.experimental.pallas import tpu as pltpu
from jax.experimental.pallas import tpu_sc as plsc

_B = 4096
_H = 200
_D = 64
_NW = 32
_TOT = _B * _H
_RPW = _TOT // _NW
_CH = 128
_NCH = _RPW // _CH
_NBUF = 4


def _sc_gather(x3, table):
    mesh = plsc.VectorSubcoreMesh(core_axis_name="c", subcore_axis_name="s")

    @functools.partial(
        pl.kernel,
        mesh=mesh,
        compiler_params=pltpu.CompilerParams(use_tc_tiling_on_sc=False),
        out_type=jax.ShapeDtypeStruct((_TOT, _D), jnp.float32),
        scratch_types=[
            pltpu.VMEM((_NCH, _CH), jnp.int32),
            *[pltpu.VMEM((_CH, _D), jnp.float32) for _ in range(_NBUF)],
            *[pltpu.SemaphoreType.DMA for _ in range(_NBUF)],
        ],
    )
    def k(x_hbm, table_hbm, out_hbm, idx_v, *rest):
        rows = rest[:_NBUF]
        sems = rest[_NBUF:]
        wid = lax.axis_index("s") * 2 + lax.axis_index("c")
        base = wid * _RPW

        pltpu.sync_copy(x_hbm.at[wid], idx_v)

        def start_gather(c, b):
            pltpu.make_async_copy(
                table_hbm.at[idx_v.at[c]], rows[b], sems[b]
            ).start()

        def wait_gather(c, b):
            pltpu.make_async_copy(
                table_hbm.at[idx_v.at[c]], rows[b], sems[b]
            ).wait()

        def drain(c, b):
            wait_gather(c, b)
            pltpu.sync_copy(rows[b], out_hbm.at[pl.ds(base + c * _CH, _CH)])

        for b in range(_NBUF):
            start_gather(b, b)

        def outer(jo, carry):
            for b in range(_NBUF):
                c = jo * _NBUF + b
                drain(c, b)
                start_gather(c + _NBUF, b)
            return carry

        lax.fori_loop(0, _NCH // _NBUF - 1, outer, 0)

        for b in range(_NBUF):
            drain(_NCH - _NBUF + b, b)

    return k(x3, table)


def kernel(x, table):
    x3 = x.reshape(_NW, _NCH, _CH)
    out = _sc_gather(x3, table)
    return out.reshape(_B, _H, _D)


# speedup vs baseline: 1.2216x; 1.2216x over previous
import functools

import jax
import jax.numpy as jnp
from jax import lax
from jax.experimental import pallas as pl
from jax.experimental.pallas import tpu as pltpu
from jax.experimental.pallas import tpu_sc as plsc

_B = 4096
_H = 200
_D = 64
_DP = 128
_NW = 32
_TOT = _B * _H
_RPW = _TOT // _NW
_CH = 128
_NCH = _RPW // _CH
_NBUF = 4


def _sc_gather(x3, tpad):
    mesh = plsc.VectorSubcoreMesh(core_axis_name="c", subcore_axis_name="s")

    @functools.partial(
        pl.kernel,
        mesh=mesh,
        out_type=jax.ShapeDtypeStruct((_TOT, _DP), jnp.float32),
        scratch_types=[
            pltpu.VMEM((_NCH, _CH), jnp.int32),
            *[pltpu.VMEM((_CH, _DP), jnp.float32) for _ in range(_NBUF)],
            *[pltpu.SemaphoreType.DMA for _ in range(_NBUF)],
        ],
    )
    def k(x_hbm, table_hbm, out_hbm, idx_v, *rest):
        rows = rest[:_NBUF]
        sems = rest[_NBUF:]
        wid = lax.axis_index("s") * 2 + lax.axis_index("c")
        base = wid * _RPW

        pltpu.sync_copy(x_hbm.at[wid], idx_v)

        def start_gather(c, b):
            pltpu.make_async_copy(
                table_hbm.at[idx_v.at[c]], rows[b], sems[b]
            ).start()

        def drain(c, b):
            pltpu.make_async_copy(
                table_hbm.at[idx_v.at[c]], rows[b], sems[b]
            ).wait()
            pltpu.sync_copy(rows[b], out_hbm.at[pl.ds(base + c * _CH, _CH)])

        for b in range(_NBUF):
            start_gather(b, b)

        def outer(jo, carry):
            for b in range(_NBUF):
                c = jo * _NBUF + b
                drain(c, b)
                start_gather(c + _NBUF, b)
            return carry

        lax.fori_loop(0, _NCH // _NBUF - 1, outer, 0)

        for b in range(_NBUF):
            drain(_NCH - _NBUF + b, b)

    return k(x3, tpad)


def kernel(x, table):
    x3 = x.reshape(_NW, _NCH, _CH)
    tpad = jnp.pad(table, ((0, 0), (0, _DP - _D)))
    out = _sc_gather(x3, tpad)
    return out[:, :_D].reshape(_B, _H, _D)


# speedup vs baseline: 1.2962x; 1.0611x over previous
import functools

import jax
import jax.numpy as jnp
from jax import lax
from jax.experimental import pallas as pl
from jax.experimental.pallas import tpu as pltpu
from jax.experimental.pallas import tpu_sc as plsc

_V = 1000000
_B = 4096
_H = 200
_D = 64
_DP = 128
_NW = 32
_TOT = _B * _H
_RPW = _TOT // _NW
_CH = 128
_NCH = _RPW // _CH
_NBUF = 4

_TC_COLS = 2048
_TC_GRID = -(-_V // _TC_COLS)


def _transpose_block(tt_ref, out_ref):
    out_ref[:, :_D] = tt_ref[...].T


def _pad_table(table_t):
    return pl.pallas_call(
        _transpose_block,
        grid=(_TC_GRID,),
        in_specs=[pl.BlockSpec((_D, _TC_COLS), lambda i: (0, i))],
        out_specs=pl.BlockSpec((_TC_COLS, _DP), lambda i: (i, 0)),
        out_shape=jax.ShapeDtypeStruct((_V, _DP), jnp.float32),
    )(table_t)


def _sc_gather(x3, tpad):
    mesh = plsc.VectorSubcoreMesh(core_axis_name="c", subcore_axis_name="s")

    @functools.partial(
        pl.kernel,
        mesh=mesh,
        out_type=jax.ShapeDtypeStruct((_TOT, _DP), jnp.float32),
        scratch_types=[
            pltpu.VMEM((_NCH, _CH), jnp.int32),
            *[pltpu.VMEM((_CH, _DP), jnp.float32) for _ in range(_NBUF)],
            *[pltpu.SemaphoreType.DMA for _ in range(_NBUF)],
        ],
    )
    def k(x_hbm, table_hbm, out_hbm, idx_v, *rest):
        rows = rest[:_NBUF]
        sems = rest[_NBUF:]
        wid = lax.axis_index("s") * 2 + lax.axis_index("c")
        base = wid * _RPW

        pltpu.sync_copy(x_hbm.at[wid], idx_v)

        def start_gather(c, b):
            pltpu.make_async_copy(
                table_hbm.at[idx_v.at[c]], rows[b], sems[b]
            ).start()

        def drain(c, b):
            pltpu.make_async_copy(
                table_hbm.at[idx_v.at[c]], rows[b], sems[b]
            ).wait()
            pltpu.sync_copy(rows[b], out_hbm.at[pl.ds(base + c * _CH, _CH)])

        for b in range(_NBUF):
            start_gather(b, b)

        def outer(jo, carry):
            for b in range(_NBUF):
                c = jo * _NBUF + b
                drain(c, b)
                start_gather(c + _NBUF, b)
            return carry

        lax.fori_loop(0, _NCH // _NBUF - 1, outer, 0)

        for b in range(_NBUF):
            drain(_NCH - _NBUF + b, b)

    return k(x3, tpad)


def kernel(x, table):
    x3 = x.reshape(_NW, _NCH, _CH)
    tpad = _pad_table(table.T)
    out = _sc_gather(x3, tpad)
    return out[:, :_D].reshape(_B, _H, _D)
